# baseline (speedup 1.0000x reference)
import functools

import jax
import jax.numpy as jnp
from jax import lax
from jax.experimental import pallas as pl
from jax.experimental.pallas import tpu as pltpu
from jax.experimental.pallas import tpu_sc as plsc

B = 2
N_IN = 12
KEEP = 3


def _sc_body(in_hbm, scores_hbm, out_a, out_b, scores_v, sem):
    wid = lax.axis_index("s") * 2 + lax.axis_index("c")

    pltpu.sync_copy(scores_hbm, scores_v)

    src = [[jnp.int32(0)] * KEEP for _ in range(B)]
    for b in range(B):
        s_vec = scores_v[b, pl.ds(0, 16)]
        s = [s_vec[i] for i in range(N_IN)]
        for j in range(N_IN):
            rank = jnp.int32(0)
            for k in range(N_IN):
                before = (s[k] < s[j]) | ((s[k] == s[j]) & (k < j))
                rank = rank + jnp.where(before, 1, 0)
            for p in range(KEEP):
                sel = rank == (KEEP + p)
                src[b][p] = jnp.where(sel, jnp.int32(j), src[b][p])

    for b in range(B):
        for p in range(KEEP):
            j = b * KEEP + p

            @pl.when(wid == 2 * j)
            def _():
                pltpu.async_copy(in_hbm.at[b, src[b][p]], out_a.at[b, p], sem).wait()

            @pl.when(wid == 2 * j + 1)
            def _():
                pltpu.async_copy(in_hbm.at[b, src[b][p]], out_b.at[j], sem).wait()


@jax.jit
def _sc_gather(image_latent, scores_padded):
    mesh = plsc.VectorSubcoreMesh(core_axis_name="c", subcore_axis_name="s")
    f = pl.kernel(
        _sc_body,
        out_type=(
            jax.ShapeDtypeStruct((B, KEEP, 384, 32, 32), jnp.float32),
            jax.ShapeDtypeStruct((B * KEEP, 384, 32, 32), jnp.float32),
        ),
        mesh=mesh,
        scratch_types=[
            pltpu.VMEM((8, 128), jnp.float32),
            pltpu.SemaphoreType.DMA,
        ],
        compiler_params=pltpu.CompilerParams(use_tc_tiling_on_sc=True),
    )
    return f(image_latent, scores_padded)


def kernel(image_latent):
    scores = jax.random.uniform(jax.random.key(42), (B, N_IN), dtype=jnp.float32)
    scores_padded = jnp.full((8, 128), 2.0, jnp.float32).at[:B, :N_IN].set(scores)
    return _sc_gather(image_latent, scores_padded)


# speedup vs baseline: 77.0968x; 77.0968x over previous
import functools

import jax
import jax.numpy as jnp
from jax import lax
from jax.experimental import pallas as pl
from jax.experimental.pallas import tpu as pltpu
from jax.experimental.pallas import tpu_sc as plsc

B = 2
N_IN = 12
KEEP = 3
H = 32


def _sc_body(in_hbm, scores_hbm, out_a, out_b, scores_v, bufs, sem_in, sem_out):
    wid = lax.axis_index("s") * 2 + lax.axis_index("c")

    pltpu.sync_copy(scores_hbm, scores_v)

    src = [[jnp.int32(0)] * KEEP for _ in range(B)]
    for b in range(B):
        s_vec = scores_v[b, pl.ds(0, 16)]
        s = [s_vec[i] for i in range(N_IN)]
        for j in range(N_IN):
            rank = jnp.int32(0)
            for k in range(N_IN):
                before = (s[k] < s[j]) | ((s[k] == s[j]) & (k < j))
                rank = rank + jnp.where(before, 1, 0)
            for p in range(KEEP):
                sel = rank == (KEEP + p)
                src[b][p] = jnp.where(sel, jnp.int32(j), src[b][p])

    gathers = [
        pltpu.async_copy(
            in_hbm.at[b, src[b][p], wid], bufs.at[b * KEEP + p], sem_in
        )
        for b in range(B)
        for p in range(KEEP)
    ]
    for g in gathers:
        g.wait()
    stores = []
    for b in range(B):
        for p in range(KEEP):
            j = b * KEEP + p
            stores.append(pltpu.async_copy(bufs.at[j], out_a.at[b, p, wid], sem_out))
            stores.append(pltpu.async_copy(bufs.at[j], out_b.at[j, wid], sem_out))
    for s_ in stores:
        s_.wait()


@jax.jit
def _sc_gather(xt, scores_padded):
    mesh = plsc.VectorSubcoreMesh(core_axis_name="c", subcore_axis_name="s")
    f = pl.kernel(
        _sc_body,
        out_type=(
            jax.ShapeDtypeStruct((B, KEEP, H, 32, 384), jnp.float32),
            jax.ShapeDtypeStruct((B * KEEP, H, 32, 384), jnp.float32),
        ),
        mesh=mesh,
        scratch_types=[
            pltpu.VMEM((8, 128), jnp.float32),
            pltpu.VMEM((B * KEEP, 32, 384), jnp.float32),
            pltpu.SemaphoreType.DMA,
            pltpu.SemaphoreType.DMA,
        ],
        compiler_params=pltpu.CompilerParams(use_tc_tiling_on_sc=True),
    )
    return f(xt, scores_padded)


def kernel(image_latent):
    scores = jax.random.uniform(jax.random.key(42), (B, N_IN), dtype=jnp.float32)
    scores_padded = jnp.full((8, 128), 2.0, jnp.float32).at[:B, :N_IN].set(scores)
    xt = jnp.transpose(image_latent, (0, 1, 3, 4, 2))
    ya, yb = _sc_gather(xt, scores_padded)
    return (
        jnp.transpose(ya, (0, 1, 4, 2, 3)),
        jnp.transpose(yb, (0, 3, 1, 2)),
    )
